# baseline (speedup 1.0000x reference)
import functools

import jax
import jax.numpy as jnp
from jax import lax
from jax.experimental import pallas as pl
from jax.experimental.pallas import tpu as pltpu
from jax.experimental.pallas import tpu_sc as plsc

N = 10000
E = 160000
D = 16
STEPS = 3

NC = 2
NS = 16
NW = NC * NS

EW = E // NW
CHUNK = 125
K = EW // CHUNK
ZR = N // NS


def _sc_gather_body(x_hbm, src_hbm, xs_hbm, idx_v, rows_v, sem):
    c = lax.axis_index("c")
    s = lax.axis_index("s")
    wid = c * NS + s
    pltpu.sync_copy(src_hbm.at[wid], idx_v)

    def fire(j, carry):
        pltpu.async_copy(x_hbm.at[idx_v.at[j]], rows_v.at[j], sem)
        return carry

    lax.fori_loop(0, K, fire, 0)

    def drain(j, carry):
        pltpu.make_async_copy(x_hbm.at[idx_v.at[j]], rows_v.at[j], sem).wait()
        return carry

    lax.fori_loop(0, K, drain, 0)
    pltpu.sync_copy(rows_v, xs_hbm.at[wid])


def _sc_scatter_body(msg_hbm, dst_hbm, out_hbm, idx_v, msg_v, zb_v, accum_sh):
    c = lax.axis_index("c")
    s = lax.axis_index("s")
    wid = c * NS + s

    def zrow(i, carry):
        zb_v[i] = jnp.zeros((D,), jnp.float32)
        return carry

    lax.fori_loop(0, ZR, zrow, 0)
    pltpu.sync_copy(zb_v, accum_sh.at[pl.ds(s * ZR, ZR)])
    pltpu.sync_copy(dst_hbm.at[wid], idx_v)
    pltpu.sync_copy(msg_hbm.at[wid], msg_v)
    plsc.subcore_barrier()

    def body(j, carry):
        pltpu.sync_copy(msg_v.at[j], accum_sh.at[idx_v.at[j]], add=True)
        return carry

    lax.fori_loop(0, K, body, 0)
    plsc.subcore_barrier()
    pltpu.sync_copy(
        accum_sh.at[pl.ds(s * ZR, ZR)], out_hbm.at[c, pl.ds(s * ZR, ZR)]
    )


def _sc_degree_body(dst_hbm, out_hbm, idx_v, ones_v, zb_v, accum_sh):
    c = lax.axis_index("c")
    s = lax.axis_index("s")
    wid = c * NS + s

    def zrow(i, carry):
        zb_v[i] = jnp.zeros((D,), jnp.float32)
        return carry

    lax.fori_loop(0, ZR, zrow, 0)

    def orow(i, carry):
        ones_v[i] = jnp.ones((D,), jnp.float32)
        return carry

    lax.fori_loop(0, CHUNK, orow, 0)
    pltpu.sync_copy(zb_v, accum_sh.at[pl.ds(s * ZR, ZR)])
    pltpu.sync_copy(dst_hbm.at[wid], idx_v)
    plsc.subcore_barrier()

    def body(j, carry):
        pltpu.sync_copy(ones_v, accum_sh.at[idx_v.at[j]], add=True)
        return carry

    lax.fori_loop(0, K, body, 0)
    plsc.subcore_barrier()
    pltpu.sync_copy(
        accum_sh.at[pl.ds(s * ZR, ZR)], out_hbm.at[c, pl.ds(s * ZR, ZR)]
    )


@functools.cache
def _sc_kernels():
    mesh = plsc.VectorSubcoreMesh(
        core_axis_name="c", subcore_axis_name="s",
        num_cores=NC, num_subcores=NS,
    )
    params = pltpu.CompilerParams(use_tc_tiling_on_sc=False)
    gather = pl.kernel(
        _sc_gather_body,
        out_type=jax.ShapeDtypeStruct((NW, K, CHUNK, D), jnp.float32),
        mesh=mesh,
        scratch_types=[
            pltpu.VMEM((K, CHUNK), jnp.int32),
            pltpu.VMEM((K, CHUNK, D), jnp.float32),
            pltpu.SemaphoreType.DMA,
        ],
        compiler_params=params,
    )
    scatter = pl.kernel(
        _sc_scatter_body,
        out_type=jax.ShapeDtypeStruct((NC, N, D), jnp.float32),
        mesh=mesh,
        scratch_types=[
            pltpu.VMEM((K, CHUNK), jnp.int32),
            pltpu.VMEM((K, CHUNK, D), jnp.float32),
            pltpu.VMEM((ZR, D), jnp.float32),
            pltpu.VMEM_SHARED((N, D), jnp.float32),
        ],
        compiler_params=params,
    )
    degree = pl.kernel(
        _sc_degree_body,
        out_type=jax.ShapeDtypeStruct((NC, N, D), jnp.float32),
        mesh=mesh,
        scratch_types=[
            pltpu.VMEM((K, CHUNK), jnp.int32),
            pltpu.VMEM((CHUNK, D), jnp.float32),
            pltpu.VMEM((ZR, D), jnp.float32),
            pltpu.VMEM_SHARED((N, D), jnp.float32),
        ],
        compiler_params=params,
    )
    return gather, scatter, degree


def _proj_body(nf_ref, wT_ref, b_ref, o_ref):
    z = jnp.dot(nf_ref[...], wT_ref[...], preferred_element_type=jnp.float32)
    o_ref[...] = jnp.maximum(z + b_ref[...], 0.0)


def _msg_body(ef_ref, xs_ref, e1wT_ref, e1b_ref, e2wT_ref, e2b_ref,
              r_ref, s_ref, o_ref):
    xs_p = xs_ref[...]
    e1wT = e1wT_ref[...].astype(jnp.bfloat16)
    e1b = e1b_ref[...]
    e2wT = e2wT_ref[...].astype(jnp.bfloat16)
    e2b = e2b_ref[...]
    rm = r_ref[...]
    sm = s_ref[...]
    rr = _EB // 8
    ef = ef_ref[...].astype(jnp.bfloat16)
    xs = jnp.concatenate([xs_p[:, D * a:D * (a + 1)] for a in range(8)],
                         axis=0)
    eh = jnp.dot(ef, e1wT, preferred_element_type=jnp.float32)
    eh = jnp.maximum(eh + e1b, 0.0).astype(jnp.bfloat16)
    we = jnp.dot(eh, e2wT, preferred_element_type=jnp.float32) + e2b
    xrep = jnp.dot(xs, rm, preferred_element_type=jnp.float32)
    msg = jnp.dot(xrep * we, sm, preferred_element_type=jnp.float32)
    for a in range(8):
        o_ref[:, D * a:D * (a + 1)] = msg[rr * a:rr * (a + 1), :]


def _post_body(p_ref, dp_ref, h_ref, cb_ref, g_ref, b_ref,
               wihT_ref, whhT_ref, bih_ref, bhh_ref, o_ref):
    summed = p_ref[0:N, :] + p_ref[N:2 * N, :]
    deg = dp_ref[0:N, :] + dp_ref[N:2 * N, :]
    rst = summed / jnp.maximum(deg, 1.0) + cb_ref[...]
    mean = jnp.mean(rst, axis=0, keepdims=True)
    var = jnp.mean((rst - mean) ** 2, axis=0, keepdims=True)
    rst = (rst - mean) * lax.rsqrt(var + 1e-5) * g_ref[...] + b_ref[...]
    xn = jnp.maximum(rst, 0.0)
    h = h_ref[...]
    gi = jnp.dot(xn, wihT_ref[...], preferred_element_type=jnp.float32)
    gi = gi + bih_ref[...]
    gh = jnp.dot(h, whhT_ref[...], preferred_element_type=jnp.float32)
    gh = gh + bhh_ref[...]
    r = jax.nn.sigmoid(gi[:, 0:D] + gh[:, 0:D])
    z = jax.nn.sigmoid(gi[:, D:2 * D] + gh[:, D:2 * D])
    nn_ = jnp.tanh(gi[:, 2 * D:3 * D] + r * gh[:, 2 * D:3 * D])
    o_ref[...] = (1.0 - z) * nn_ + z * h


_EB = 8000


def _msg_call(edge_feats, xs, e1wT, e1b, e2wT, e2b, rmat, smat):
    d_eh = e1wT.shape[1]
    grid = (E // _EB,)
    return pl.pallas_call(
        _msg_body,
        grid=grid,
        in_specs=[
            pl.BlockSpec((_EB, D), lambda i: (i, 0)),
            pl.BlockSpec((_EB // 8, 128), lambda i: (i, 0)),
            pl.BlockSpec((D, d_eh), lambda i: (0, 0)),
            pl.BlockSpec((1, d_eh), lambda i: (0, 0)),
            pl.BlockSpec((d_eh, D * D), lambda i: (0, 0)),
            pl.BlockSpec((1, D * D), lambda i: (0, 0)),
            pl.BlockSpec((D, D * D), lambda i: (0, 0)),
            pl.BlockSpec((D * D, D), lambda i: (0, 0)),
        ],
        out_specs=pl.BlockSpec((_EB // 8, 128), lambda i: (i, 0)),
        out_shape=jax.ShapeDtypeStruct((E // 8, 128), jnp.float32),
    )(edge_feats, xs, e1wT, e1b, e2wT, e2b, rmat, smat)


def kernel(node_feats, edge_feats, edge_index, proj_W, proj_b, e1_W, e1_b,
           e2_W, e2_b, conv_bias, bn_gamma, bn_beta,
           gru_W_ih, gru_W_hh, gru_b_ih, gru_b_hh):
    sc_gather, sc_scatter, sc_degree = _sc_kernels()
    def slot_order(ix):
        return (ix.reshape(E // _EB, 8, _EB // 8)
                .transpose(0, 2, 1).reshape(NW, K, CHUNK))

    src = slot_order(edge_index[0])
    dst = slot_order(edge_index[1])

    projT = proj_W.T
    e1wT = e1_W.T
    e2wT = e2_W.T
    wihT = gru_W_ih.T
    whhT = gru_W_hh.T
    rmat = jnp.kron(jnp.eye(D, dtype=jnp.float32),
                    jnp.ones((1, D), jnp.float32))
    smat = jnp.kron(jnp.ones((D, 1), jnp.float32),
                    jnp.eye(D, dtype=jnp.float32))

    x = pl.pallas_call(
        _proj_body,
        out_shape=jax.ShapeDtypeStruct((N, D), jnp.float32),
    )(node_feats, projT, proj_b.reshape(1, D))

    degp = sc_degree(dst).reshape(2 * N, D)

    post = pl.pallas_call(
        _post_body,
        out_shape=jax.ShapeDtypeStruct((N, D), jnp.float32),
    )

    for _ in range(STEPS):
        xs = sc_gather(x, src).reshape(E // 8, 128)
        msg = _msg_call(edge_feats, xs, e1wT, e1_b.reshape(1, -1),
                        e2wT, e2_b.reshape(1, -1), rmat, smat)
        prt = sc_scatter(msg.reshape(NW, K, CHUNK, D), dst)
        x = post(prt.reshape(2 * N, D), degp, x,
                 conv_bias.reshape(1, D), bn_gamma.reshape(1, D),
                 bn_beta.reshape(1, D), wihT, whhT,
                 gru_b_ih.reshape(1, 3 * D), gru_b_hh.reshape(1, 3 * D))
    return x


# speedup vs baseline: 1.2003x; 1.2003x over previous
import functools

import jax
import jax.numpy as jnp
from jax import lax
from jax.experimental import pallas as pl
from jax.experimental.pallas import tpu as pltpu
from jax.experimental.pallas import tpu_sc as plsc

N = 10000
E = 160000
D = 16
STEPS = 3

NC = 2
NS = 16
NW = NC * NS

EW = E // NW
CHUNK = 125
K = EW // CHUNK
ZR = N // NS


def _sc_gather_body(x_hbm, src_hbm, xs_hbm, idx_v, rows_v, sem):
    c = lax.axis_index("c")
    s = lax.axis_index("s")
    wid = c * NS + s
    pltpu.sync_copy(src_hbm.at[wid], idx_v)

    def fire(j, carry):
        pltpu.async_copy(x_hbm.at[idx_v.at[j]], rows_v.at[j], sem)
        return carry

    lax.fori_loop(0, K, fire, 0)

    def drain(j, carry):
        pltpu.make_async_copy(x_hbm.at[idx_v.at[j]], rows_v.at[j], sem).wait()
        return carry

    lax.fori_loop(0, K, drain, 0)
    pltpu.sync_copy(rows_v, xs_hbm.at[wid])


def _sc_scatter_body(msg_hbm, dst_hbm, out_hbm, idx_v, msg_v, zb_v, accum_sh):
    c = lax.axis_index("c")
    s = lax.axis_index("s")
    wid = c * NS + s

    def zrow(i, carry):
        zb_v[i] = jnp.zeros((D,), jnp.float32)
        return carry

    lax.fori_loop(0, ZR, zrow, 0)
    pltpu.sync_copy(zb_v, accum_sh.at[pl.ds(s * ZR, ZR)])
    pltpu.sync_copy(dst_hbm.at[wid], idx_v)
    pltpu.sync_copy(msg_hbm.at[wid], msg_v)
    plsc.subcore_barrier()

    def body(j, carry):
        pltpu.sync_copy(msg_v.at[j], accum_sh.at[idx_v.at[j]], add=True)
        return carry

    lax.fori_loop(0, K, body, 0)
    plsc.subcore_barrier()
    pltpu.sync_copy(
        accum_sh.at[pl.ds(s * ZR, ZR)], out_hbm.at[c, pl.ds(s * ZR, ZR)]
    )


def _sc_degree_body(dst_hbm, out_hbm, idx_v, ones_v, zb_v, accum_sh):
    c = lax.axis_index("c")
    s = lax.axis_index("s")
    wid = c * NS + s

    def zrow(i, carry):
        zb_v[i] = jnp.zeros((D,), jnp.float32)
        return carry

    lax.fori_loop(0, ZR, zrow, 0)

    def orow(i, carry):
        ones_v[i] = jnp.ones((D,), jnp.float32)
        return carry

    lax.fori_loop(0, CHUNK, orow, 0)
    pltpu.sync_copy(zb_v, accum_sh.at[pl.ds(s * ZR, ZR)])
    pltpu.sync_copy(dst_hbm.at[wid], idx_v)
    plsc.subcore_barrier()

    def body(j, carry):
        pltpu.sync_copy(ones_v, accum_sh.at[idx_v.at[j]], add=True)
        return carry

    lax.fori_loop(0, K, body, 0)
    plsc.subcore_barrier()
    pltpu.sync_copy(
        accum_sh.at[pl.ds(s * ZR, ZR)], out_hbm.at[c, pl.ds(s * ZR, ZR)]
    )


@functools.cache
def _sc_kernels():
    mesh = plsc.VectorSubcoreMesh(
        core_axis_name="c", subcore_axis_name="s",
        num_cores=NC, num_subcores=NS,
    )
    params = pltpu.CompilerParams(use_tc_tiling_on_sc=False)
    gather = pl.kernel(
        _sc_gather_body,
        out_type=jax.ShapeDtypeStruct((NW, K, CHUNK, D), jnp.float32),
        mesh=mesh,
        scratch_types=[
            pltpu.VMEM((K, CHUNK), jnp.int32),
            pltpu.VMEM((K, CHUNK, D), jnp.float32),
            pltpu.SemaphoreType.DMA,
        ],
        compiler_params=params,
    )
    scatter = pl.kernel(
        _sc_scatter_body,
        out_type=jax.ShapeDtypeStruct((NC, N, D), jnp.float32),
        mesh=mesh,
        scratch_types=[
            pltpu.VMEM((K, CHUNK), jnp.int32),
            pltpu.VMEM((K, CHUNK, D), jnp.float32),
            pltpu.VMEM((ZR, D), jnp.float32),
            pltpu.VMEM_SHARED((N, D), jnp.float32),
        ],
        compiler_params=params,
    )
    degree = pl.kernel(
        _sc_degree_body,
        out_type=jax.ShapeDtypeStruct((NC, N, D), jnp.float32),
        mesh=mesh,
        scratch_types=[
            pltpu.VMEM((K, CHUNK), jnp.int32),
            pltpu.VMEM((CHUNK, D), jnp.float32),
            pltpu.VMEM((ZR, D), jnp.float32),
            pltpu.VMEM_SHARED((N, D), jnp.float32),
        ],
        compiler_params=params,
    )
    return gather, scatter, degree


def _proj_body(nf_ref, wT_ref, b_ref, o_ref):
    z = jnp.dot(nf_ref[...], wT_ref[...], preferred_element_type=jnp.float32)
    x0 = jnp.maximum(z + b_ref[...], 0.0)
    qr = N // 8
    for a in range(8):
        o_ref[:, D * a:D * (a + 1)] = x0[qr * a:qr * (a + 1), :]


def _msg_body(ef_ref, xs_ref, e1wT_ref, e1b_ref, e2wT_ref, e2b_ref,
              r_ref, s_ref, o_ref):
    ef_p = ef_ref[...]
    xs_p = xs_ref[...]
    e1wT = e1wT_ref[...].astype(jnp.bfloat16)
    e1b = e1b_ref[...]
    e2wT = e2wT_ref[...].astype(jnp.bfloat16)
    e2b = e2b_ref[...]
    rm = r_ref[...]
    sm = s_ref[...]
    rr = _EB // 8
    ef = jnp.concatenate([ef_p[:, D * a:D * (a + 1)] for a in range(8)],
                         axis=0).astype(jnp.bfloat16)
    xs = jnp.concatenate([xs_p[:, D * a:D * (a + 1)] for a in range(8)],
                         axis=0)
    eh = jnp.dot(ef, e1wT, preferred_element_type=jnp.float32)
    eh = jnp.maximum(eh + e1b, 0.0).astype(jnp.bfloat16)
    we = jnp.dot(eh, e2wT, preferred_element_type=jnp.float32) + e2b
    xrep = jnp.dot(xs, rm, preferred_element_type=jnp.float32)
    msg = jnp.dot(xrep * we, sm, preferred_element_type=jnp.float32)
    for a in range(8):
        o_ref[:, D * a:D * (a + 1)] = msg[rr * a:rr * (a + 1), :]


def _post_body(p_ref, dp_ref, h_ref, cb_ref, g_ref, b_ref,
               wihB_ref, whhB_ref, bih_ref, bhh_ref,
               fold_ref, rep_ref, o_ref):
    qr = N // 8
    summed = p_ref[0:qr, :] + p_ref[qr:2 * qr, :]
    deg = dp_ref[0:qr, :] + dp_ref[qr:2 * qr, :]
    rst = summed / jnp.maximum(deg, 1.0) + cb_ref[...]
    colsum = jnp.sum(rst, axis=0, keepdims=True)
    mean16 = jnp.dot(colsum, fold_ref[...],
                     preferred_element_type=jnp.float32) * (1.0 / N)
    mean = jnp.dot(mean16, rep_ref[...], preferred_element_type=jnp.float32)
    dev = rst - mean
    colsq = jnp.sum(dev * dev, axis=0, keepdims=True)
    var16 = jnp.dot(colsq, fold_ref[...],
                    preferred_element_type=jnp.float32) * (1.0 / N)
    var = jnp.dot(var16, rep_ref[...], preferred_element_type=jnp.float32)
    rst = dev * lax.rsqrt(var + 1e-5) * g_ref[...] + b_ref[...]
    xn = jnp.maximum(rst, 0.0)
    h = h_ref[...]
    gi = jnp.dot(xn, wihB_ref[...], preferred_element_type=jnp.float32)
    gi = gi + bih_ref[...]
    gh = jnp.dot(h, whhB_ref[...], preferred_element_type=jnp.float32)
    gh = gh + bhh_ref[...]
    for a in range(8):
        gia = gi[:, 3 * D * a:3 * D * (a + 1)]
        gha = gh[:, 3 * D * a:3 * D * (a + 1)]
        ha = h[:, D * a:D * (a + 1)]
        r = jax.nn.sigmoid(gia[:, 0:D] + gha[:, 0:D])
        z = jax.nn.sigmoid(gia[:, D:2 * D] + gha[:, D:2 * D])
        nn_ = jnp.tanh(gia[:, 2 * D:3 * D] + r * gha[:, 2 * D:3 * D])
        o_ref[:, D * a:D * (a + 1)] = (1.0 - z) * nn_ + z * ha


_EB = 8000


def _msg_call(edge_feats, xs, e1wT, e1b, e2wT, e2b, rmat, smat):
    d_eh = e1wT.shape[1]
    grid = (E // _EB,)
    return pl.pallas_call(
        _msg_body,
        grid=grid,
        in_specs=[
            pl.BlockSpec((_EB // 8, 128), lambda i: (i, 0)),
            pl.BlockSpec((_EB // 8, 128), lambda i: (i, 0)),
            pl.BlockSpec((D, d_eh), lambda i: (0, 0)),
            pl.BlockSpec((1, d_eh), lambda i: (0, 0)),
            pl.BlockSpec((d_eh, D * D), lambda i: (0, 0)),
            pl.BlockSpec((1, D * D), lambda i: (0, 0)),
            pl.BlockSpec((D, D * D), lambda i: (0, 0)),
            pl.BlockSpec((D * D, D), lambda i: (0, 0)),
        ],
        out_specs=pl.BlockSpec((_EB // 8, 128), lambda i: (i, 0)),
        out_shape=jax.ShapeDtypeStruct((E // 8, 128), jnp.float32),
    )(edge_feats, xs, e1wT, e1b, e2wT, e2b, rmat, smat)


def kernel(node_feats, edge_feats, edge_index, proj_W, proj_b, e1_W, e1_b,
           e2_W, e2_b, conv_bias, bn_gamma, bn_beta,
           gru_W_ih, gru_W_hh, gru_b_ih, gru_b_hh):
    sc_gather, sc_scatter, sc_degree = _sc_kernels()
    src = edge_index[0].reshape(NW, K, CHUNK)
    dst = edge_index[1].reshape(NW, K, CHUNK)

    projT = proj_W.T
    e1wT = e1_W.T
    e2wT = e2_W.T
    rmat = jnp.kron(jnp.eye(D, dtype=jnp.float32),
                    jnp.ones((1, D), jnp.float32))
    smat = jnp.kron(jnp.ones((D, 1), jnp.float32),
                    jnp.eye(D, dtype=jnp.float32))
    fold = jnp.kron(jnp.ones((8, 1), jnp.float32),
                    jnp.eye(D, dtype=jnp.float32))
    rep = jnp.kron(jnp.ones((1, 8), jnp.float32),
                   jnp.eye(D, dtype=jnp.float32))
    wihB = jnp.kron(jnp.eye(8, dtype=jnp.float32), gru_W_ih.T)
    whhB = jnp.kron(jnp.eye(8, dtype=jnp.float32), gru_W_hh.T)
    bihB = jnp.tile(gru_b_ih, 8).reshape(1, 24 * D)
    bhhB = jnp.tile(gru_b_hh, 8).reshape(1, 24 * D)
    cbB = jnp.tile(conv_bias, 8).reshape(1, 8 * D)
    gB = jnp.tile(bn_gamma, 8).reshape(1, 8 * D)
    bB = jnp.tile(bn_beta, 8).reshape(1, 8 * D)

    nf_perm = (node_feats.reshape(N // 8, 8, node_feats.shape[1])
               .transpose(1, 0, 2).reshape(N, node_feats.shape[1]))
    x = pl.pallas_call(
        _proj_body,
        out_shape=jax.ShapeDtypeStruct((N // 8, 8 * D), jnp.float32),
    )(nf_perm, projT, proj_b.reshape(1, D))

    degp = sc_degree(dst).reshape(N // 4, 8 * D)
    ef_p = edge_feats.reshape(E // 8, 8 * D)

    post = pl.pallas_call(
        _post_body,
        out_shape=jax.ShapeDtypeStruct((N // 8, 8 * D), jnp.float32),
    )

    for _ in range(STEPS):
        xs = sc_gather(x.reshape(N, D), src).reshape(E // 8, 128)
        msg = _msg_call(ef_p, xs, e1wT, e1_b.reshape(1, -1),
                        e2wT, e2_b.reshape(1, -1), rmat, smat)
        prt = sc_scatter(msg.reshape(NW, K, CHUNK, D), dst)
        x = post(prt.reshape(N // 4, 8 * D), degp, x,
                 cbB, gB, bB, wihB, whhB, bihB, bhhB, fold, rep)
    return x.reshape(N, D)


# speedup vs baseline: 1.2193x; 1.0158x over previous
import functools

import jax
import jax.numpy as jnp
from jax import lax
from jax.experimental import pallas as pl
from jax.experimental.pallas import tpu as pltpu
from jax.experimental.pallas import tpu_sc as plsc

N = 10000
E = 160000
D = 16
STEPS = 3

NC = 2
NS = 16
NW = NC * NS

EW = E // NW
CHUNK = 125
K = EW // CHUNK
ZR = N // NS


def _sc_gather_body(x_hbm, src_hbm, xs_hbm, idx_v, rows_v, sem):
    c = lax.axis_index("c")
    s = lax.axis_index("s")
    wid = c * NS + s
    pltpu.sync_copy(src_hbm.at[wid], idx_v)

    def fire(j, carry):
        pltpu.async_copy(x_hbm.at[idx_v.at[j]], rows_v.at[j], sem)
        return carry

    lax.fori_loop(0, K, fire, 0)

    def drain(j, carry):
        pltpu.make_async_copy(x_hbm.at[idx_v.at[j]], rows_v.at[j], sem).wait()
        return carry

    lax.fori_loop(0, K, drain, 0)
    pltpu.sync_copy(rows_v, xs_hbm.at[wid])


def _sc_scatter_body(msg_hbm, dst_hbm, out_hbm, idx_v, msg_v, zb_v, accum_sh):
    c = lax.axis_index("c")
    s = lax.axis_index("s")
    wid = c * NS + s

    def zrow(i, carry):
        zb_v[i] = jnp.zeros((D,), jnp.float32)
        return carry

    lax.fori_loop(0, ZR, zrow, 0)
    pltpu.sync_copy(zb_v, accum_sh.at[pl.ds(s * ZR, ZR)])
    pltpu.sync_copy(dst_hbm.at[wid], idx_v)
    pltpu.sync_copy(msg_hbm.at[wid], msg_v)
    plsc.subcore_barrier()

    def body(j, carry):
        pltpu.sync_copy(msg_v.at[j], accum_sh.at[idx_v.at[j]], add=True)
        return carry

    lax.fori_loop(0, K, body, 0)
    plsc.subcore_barrier()
    pltpu.sync_copy(
        accum_sh.at[pl.ds(s * ZR, ZR)], out_hbm.at[c, pl.ds(s * ZR, ZR)]
    )


def _sc_degree_body(dst_hbm, out_hbm, idx_v, ones_v, zb_v, accum_sh):
    c = lax.axis_index("c")
    s = lax.axis_index("s")
    wid = c * NS + s

    def zrow(i, carry):
        zb_v[i] = jnp.zeros((D,), jnp.float32)
        return carry

    lax.fori_loop(0, ZR, zrow, 0)

    def orow(i, carry):
        ones_v[i] = jnp.ones((D,), jnp.float32)
        return carry

    lax.fori_loop(0, CHUNK, orow, 0)
    pltpu.sync_copy(zb_v, accum_sh.at[pl.ds(s * ZR, ZR)])
    pltpu.sync_copy(dst_hbm.at[wid], idx_v)
    plsc.subcore_barrier()

    def body(j, carry):
        pltpu.sync_copy(ones_v, accum_sh.at[idx_v.at[j]], add=True)
        return carry

    lax.fori_loop(0, K, body, 0)
    plsc.subcore_barrier()
    pltpu.sync_copy(
        accum_sh.at[pl.ds(s * ZR, ZR)], out_hbm.at[c, pl.ds(s * ZR, ZR)]
    )


@functools.cache
def _sc_kernels():
    mesh = plsc.VectorSubcoreMesh(
        core_axis_name="c", subcore_axis_name="s",
        num_cores=NC, num_subcores=NS,
    )
    params = pltpu.CompilerParams(use_tc_tiling_on_sc=False)
    gather = pl.kernel(
        _sc_gather_body,
        out_type=jax.ShapeDtypeStruct((NW, K, CHUNK, D), jnp.float32),
        mesh=mesh,
        scratch_types=[
            pltpu.VMEM((K, CHUNK), jnp.int32),
            pltpu.VMEM((K, CHUNK, D), jnp.float32),
            pltpu.SemaphoreType.DMA,
        ],
        compiler_params=params,
    )
    scatter = pl.kernel(
        _sc_scatter_body,
        out_type=jax.ShapeDtypeStruct((NC, N, D), jnp.float32),
        mesh=mesh,
        scratch_types=[
            pltpu.VMEM((K, CHUNK), jnp.int32),
            pltpu.VMEM((K, CHUNK, D), jnp.float32),
            pltpu.VMEM((ZR, D), jnp.float32),
            pltpu.VMEM_SHARED((N, D), jnp.float32),
        ],
        compiler_params=params,
    )
    degree = pl.kernel(
        _sc_degree_body,
        out_type=jax.ShapeDtypeStruct((NC, N, D), jnp.float32),
        mesh=mesh,
        scratch_types=[
            pltpu.VMEM((K, CHUNK), jnp.int32),
            pltpu.VMEM((CHUNK, D), jnp.float32),
            pltpu.VMEM((ZR, D), jnp.float32),
            pltpu.VMEM_SHARED((N, D), jnp.float32),
        ],
        compiler_params=params,
    )
    return gather, scatter, degree


def _proj_body(nf_ref, wT_ref, b_ref, o_ref):
    z = jnp.dot(nf_ref[...], wT_ref[...], preferred_element_type=jnp.float32)
    x0 = jnp.maximum(z + b_ref[...], 0.0)
    qr = N // 8
    for a in range(8):
        o_ref[:, D * a:D * (a + 1)] = x0[qr * a:qr * (a + 1), :]


def _msg_body(ef_ref, xs_ref, e1wT_ref, e1b_ref, e2wT_ref, e2b_ref,
              r_ref, s_ref, o_ref):
    ef_p = ef_ref[...]
    xs_p = xs_ref[...]
    e1wT = e1wT_ref[...].astype(jnp.bfloat16)
    e1b = e1b_ref[...]
    e2wT = e2wT_ref[...].astype(jnp.bfloat16)
    e2b = e2b_ref[...]
    rm = r_ref[...]
    sm = s_ref[...]
    rr = _EB // 8
    ef = jnp.concatenate([ef_p[:, D * a:D * (a + 1)] for a in range(8)],
                         axis=0).astype(jnp.bfloat16)
    xs = jnp.concatenate([xs_p[:, D * a:D * (a + 1)] for a in range(8)],
                         axis=0)
    eh = jnp.dot(ef, e1wT, preferred_element_type=jnp.float32)
    eh = jnp.maximum(eh + e1b, 0.0).astype(jnp.bfloat16)
    we = jnp.dot(eh, e2wT, preferred_element_type=jnp.float32) + e2b
    xrep = jnp.dot(xs, rm, preferred_element_type=jnp.float32)
    msg = jnp.dot(xrep * we, sm, preferred_element_type=jnp.float32)
    for a in range(8):
        o_ref[:, D * a:D * (a + 1)] = msg[rr * a:rr * (a + 1), :]


def _post_body(p_ref, dp_ref, h_ref, cb_ref, g_ref, b_ref,
               wihB_ref, whhB_ref, bih_ref, bhh_ref,
               fold_ref, rep_ref, o_ref):
    qr = N // 8
    summed = p_ref[0:qr, :] + p_ref[qr:2 * qr, :]
    deg = dp_ref[0:qr, :] + dp_ref[qr:2 * qr, :]
    rst = summed / jnp.maximum(deg, 1.0) + cb_ref[...]
    colsum = jnp.sum(rst, axis=0, keepdims=True)
    mean16 = jnp.dot(colsum, fold_ref[...],
                     preferred_element_type=jnp.float32) * (1.0 / N)
    mean = jnp.dot(mean16, rep_ref[...], preferred_element_type=jnp.float32)
    dev = rst - mean
    colsq = jnp.sum(dev * dev, axis=0, keepdims=True)
    var16 = jnp.dot(colsq, fold_ref[...],
                    preferred_element_type=jnp.float32) * (1.0 / N)
    var = jnp.dot(var16, rep_ref[...], preferred_element_type=jnp.float32)
    rst = dev * lax.rsqrt(var + 1e-5) * g_ref[...] + b_ref[...]
    xn = jnp.maximum(rst, 0.0)
    h = h_ref[...]
    gi = jnp.dot(xn, wihB_ref[...], preferred_element_type=jnp.float32)
    gi = gi + bih_ref[...]
    gh = jnp.dot(h, whhB_ref[...], preferred_element_type=jnp.float32)
    gh = gh + bhh_ref[...]
    for a in range(8):
        gia = gi[:, 3 * D * a:3 * D * (a + 1)]
        gha = gh[:, 3 * D * a:3 * D * (a + 1)]
        ha = h[:, D * a:D * (a + 1)]
        r = jax.nn.sigmoid(gia[:, 0:D] + gha[:, 0:D])
        z = jax.nn.sigmoid(gia[:, D:2 * D] + gha[:, D:2 * D])
        nn_ = jnp.tanh(gia[:, 2 * D:3 * D] + r * gha[:, 2 * D:3 * D])
        o_ref[:, D * a:D * (a + 1)] = (1.0 - z) * nn_ + z * ha


_EB = 16000


def _msg_call(edge_feats, xs, e1wT, e1b, e2wT, e2b, rmat, smat):
    d_eh = e1wT.shape[1]
    grid = (E // _EB,)
    return pl.pallas_call(
        _msg_body,
        grid=grid,
        in_specs=[
            pl.BlockSpec((_EB // 8, 128), lambda i: (i, 0)),
            pl.BlockSpec((_EB // 8, 128), lambda i: (i, 0)),
            pl.BlockSpec((D, d_eh), lambda i: (0, 0)),
            pl.BlockSpec((1, d_eh), lambda i: (0, 0)),
            pl.BlockSpec((d_eh, D * D), lambda i: (0, 0)),
            pl.BlockSpec((1, D * D), lambda i: (0, 0)),
            pl.BlockSpec((D, D * D), lambda i: (0, 0)),
            pl.BlockSpec((D * D, D), lambda i: (0, 0)),
        ],
        out_specs=pl.BlockSpec((_EB // 8, 128), lambda i: (i, 0)),
        out_shape=jax.ShapeDtypeStruct((E // 8, 128), jnp.float32),
    )(edge_feats, xs, e1wT, e1b, e2wT, e2b, rmat, smat)


def kernel(node_feats, edge_feats, edge_index, proj_W, proj_b, e1_W, e1_b,
           e2_W, e2_b, conv_bias, bn_gamma, bn_beta,
           gru_W_ih, gru_W_hh, gru_b_ih, gru_b_hh):
    sc_gather, sc_scatter, sc_degree = _sc_kernels()
    src = edge_index[0].reshape(NW, K, CHUNK)
    dst = edge_index[1].reshape(NW, K, CHUNK)

    projT = proj_W.T
    e1wT = e1_W.T
    e2wT = e2_W.T
    rmat = jnp.kron(jnp.eye(D, dtype=jnp.float32),
                    jnp.ones((1, D), jnp.float32))
    smat = jnp.kron(jnp.ones((D, 1), jnp.float32),
                    jnp.eye(D, dtype=jnp.float32))
    fold = jnp.kron(jnp.ones((8, 1), jnp.float32),
                    jnp.eye(D, dtype=jnp.float32))
    rep = jnp.kron(jnp.ones((1, 8), jnp.float32),
                   jnp.eye(D, dtype=jnp.float32))
    wihB = jnp.kron(jnp.eye(8, dtype=jnp.float32), gru_W_ih.T)
    whhB = jnp.kron(jnp.eye(8, dtype=jnp.float32), gru_W_hh.T)
    bihB = jnp.tile(gru_b_ih, 8).reshape(1, 24 * D)
    bhhB = jnp.tile(gru_b_hh, 8).reshape(1, 24 * D)
    cbB = jnp.tile(conv_bias, 8).reshape(1, 8 * D)
    gB = jnp.tile(bn_gamma, 8).reshape(1, 8 * D)
    bB = jnp.tile(bn_beta, 8).reshape(1, 8 * D)

    nf_perm = (node_feats.reshape(N // 8, 8, node_feats.shape[1])
               .transpose(1, 0, 2).reshape(N, node_feats.shape[1]))
    x = pl.pallas_call(
        _proj_body,
        out_shape=jax.ShapeDtypeStruct((N // 8, 8 * D), jnp.float32),
    )(nf_perm, projT, proj_b.reshape(1, D))

    degp = sc_degree(dst).reshape(N // 4, 8 * D)
    ef_p = edge_feats.reshape(E // 8, 8 * D)

    post = pl.pallas_call(
        _post_body,
        out_shape=jax.ShapeDtypeStruct((N // 8, 8 * D), jnp.float32),
    )

    for _ in range(STEPS):
        xs = sc_gather(x.reshape(N, D), src).reshape(E // 8, 128)
        msg = _msg_call(ef_p, xs, e1wT, e1_b.reshape(1, -1),
                        e2wT, e2_b.reshape(1, -1), rmat, smat)
        prt = sc_scatter(msg.reshape(NW, K, CHUNK, D), dst)
        x = post(prt.reshape(N // 4, 8 * D), degp, x,
                 cbB, gB, bB, wihB, whhB, bihB, bhhB, fold, rep)
    return x.reshape(N, D)
